# baseline (speedup 1.0000x reference)
import functools

import jax
import jax.numpy as jnp
from jax import lax
from jax.experimental import pallas as pl
from jax.experimental.pallas import tpu as pltpu
from jax.experimental.pallas import tpu_sc as plsc

TILE = 128
NW = 32


def _router_body(nslot, ntiles, x_ref, gwt_ref, topw_ref, slots_ref, meta_ref,
                 aux_ref, xb_ref):
    T, H = x_ref.shape
    E = gwt_ref.shape[1]
    x = x_ref[...]
    xb_ref[...] = x.astype(jnp.bfloat16)
    logits = jnp.dot(x, gwt_ref[...], preferred_element_type=jnp.float32)
    RB = 128
    m = jnp.max(logits, axis=1, keepdims=True)
    ex = jnp.exp(logits - m)
    probs = ex / jnp.sum(ex, axis=1, keepdims=True)

    e_iota = lax.broadcasted_iota(jnp.int32, (T, E), 1)
    p1 = jnp.max(probs, axis=1, keepdims=True)
    first1 = jnp.min(jnp.where(probs == p1, e_iota, E), axis=1, keepdims=True)
    masked = jnp.where(e_iota == first1, -jnp.inf, probs)
    p2 = jnp.max(masked, axis=1, keepdims=True)
    first2 = jnp.min(jnp.where(masked == p2, e_iota, E), axis=1, keepdims=True)
    s = p1 + p2
    topw_ref[0, :] = (p1 / s).reshape(T)
    topw_ref[1, :] = (p2 / s).reshape(T)

    oh1 = (e_iota == first1).astype(jnp.float32)
    oh2 = (e_iota == first2).astype(jnp.float32)
    ohs = jnp.concatenate([oh1, oh2], 0)
    ri = lax.broadcasted_iota(jnp.int32, (RB, RB), 0)
    ci = lax.broadcasted_iota(jnp.int32, (RB, RB), 1)
    ls = (ri > ci).astype(jnp.float32)
    off = jnp.zeros((1, E), jnp.float32)
    rank_parts = []
    for b in range(2 * T // RB):
        blk = lax.slice(ohs, (b * RB, 0), ((b + 1) * RB, E))
        rblk = jnp.dot(ls, blk, preferred_element_type=jnp.float32) + off
        rank_parts.append(jnp.sum(rblk * blk, axis=1))
        off = off + jnp.sum(blk, axis=0, keepdims=True)
    ranks = jnp.concatenate(rank_parts, 0)

    cnt = off[0]
    cnt_pad = jnp.ceil(cnt / TILE) * TILE
    ri16 = lax.broadcasted_iota(jnp.int32, (E, E), 0)
    ci16 = lax.broadcasted_iota(jnp.int32, (E, E), 1)
    incl = jnp.sum(jnp.where(ci16 <= ri16, cnt_pad[None, :], 0.0), axis=1)
    excl = incl - cnt_pad
    slot_f = ranks + jnp.sum(ohs * excl[None, :], axis=1)
    slots_ref[0, :] = lax.slice(slot_f, (0,), (T,)).astype(jnp.int32)
    slots_ref[1, :] = lax.slice(slot_f, (T,), (2 * T,)).astype(jnp.int32)

    total = jnp.sum(cnt_pad)
    used = total / TILE
    js = lax.broadcasted_iota(jnp.int32, (ntiles, 1), 0).astype(jnp.float32) * TILE
    slot_eff = jnp.minimum(js, total - TILE)
    te = jnp.sum((slot_eff >= incl[None, :]).astype(jnp.int32), axis=1)
    pad = meta_ref.shape[1] - ntiles - 1
    meta_ref[0, :] = jnp.concatenate(
        [te, used.astype(jnp.int32).reshape(1),
         jnp.zeros((pad,), jnp.int32)], 0)

    meanprob = jnp.mean(probs, axis=0)
    aux_ref[...] = jnp.reshape(jnp.sum(cnt * meanprob) * E / T, (1, 1))


def _router(x, gwt, nslot, ntiles, meta_len):
    T, _ = x.shape
    return pl.pallas_call(
        functools.partial(_router_body, nslot, ntiles),
        out_shape=(
            jax.ShapeDtypeStruct((2, T), jnp.float32),
            jax.ShapeDtypeStruct((2, T), jnp.int32),
            jax.ShapeDtypeStruct((1, meta_len), jnp.int32),
            jax.ShapeDtypeStruct((1, 1), jnp.float32),
            jax.ShapeDtypeStruct(x.shape, jnp.bfloat16),
        ),
    )(x, gwt)


def _make_dispatch(T, W, A, nslot):
    tpw = T // NW
    mesh = plsc.VectorSubcoreMesh(core_axis_name="c", subcore_axis_name="s")

    @functools.partial(
        pl.kernel,
        out_type=jax.ShapeDtypeStruct((nslot, W), jnp.int32),
        mesh=mesh,
        scratch_types=[
            pltpu.VMEM((tpw,), jnp.int32),
            pltpu.VMEM((tpw,), jnp.int32),
            pltpu.VMEM((tpw, W), jnp.int32),
            pltpu.SemaphoreType.DMA,
            pltpu.SemaphoreType.DMA,
        ],
    )
    def dispatch(x_hbm, slots_hbm, xs_hbm, idx0, idx1, rows_v, sem0, sem1):
        wid = lax.axis_index("s") * 2 + lax.axis_index("c")
        tb = wid * tpw
        pltpu.sync_copy(slots_hbm.at[pl.ds(tb, tpw)], idx0)
        pltpu.sync_copy(slots_hbm.at[pl.ds(T + tb, tpw)], idx1)
        pltpu.sync_copy(x_hbm.at[pl.ds(tb, tpw)], rows_v)
        c0 = pltpu.async_copy(rows_v, xs_hbm.at[idx0], sem0)
        c1 = pltpu.async_copy(rows_v, xs_hbm.at[idx1], sem1)
        c0.wait()
        c1.wait()

    return dispatch


def _gemm_body(ntiles, meta_ref, xs_ref, w1_ref, w3_ref, w2_ref, ys_ref):
    j = pl.program_id(0)

    @pl.when(j < meta_ref[0, ntiles])
    def _():
        xt = xs_ref[...].astype(jnp.float32)
        h1 = jnp.dot(xt, w1_ref[0], preferred_element_type=jnp.float32)
        h3 = jnp.dot(xt, w3_ref[0], preferred_element_type=jnp.float32)
        act = h1 * lax.logistic(h1) * h3
        ys_ref[...] = jnp.dot(act, w2_ref[0], preferred_element_type=jnp.float32)


def _gemm(meta, xs, w1, w3, w2, nslot, ntiles):
    E, H, F = w1.shape
    grid_spec = pltpu.PrefetchScalarGridSpec(
        num_scalar_prefetch=1,
        grid=(ntiles,),
        in_specs=[
            pl.BlockSpec((TILE, H),
                         lambda j, m: (jnp.minimum(j, m[0, ntiles] - 1), 0)),
            pl.BlockSpec((1, H, F), lambda j, m: (m[0, j], 0, 0)),
            pl.BlockSpec((1, H, F), lambda j, m: (m[0, j], 0, 0)),
            pl.BlockSpec((1, F, H), lambda j, m: (m[0, j], 0, 0)),
        ],
        out_specs=pl.BlockSpec(
            (TILE, H), lambda j, m: (jnp.minimum(j, m[0, ntiles] - 1), 0)),
    )
    return pl.pallas_call(
        functools.partial(_gemm_body, ntiles),
        grid_spec=grid_spec,
        out_shape=jax.ShapeDtypeStruct((nslot, H), jnp.float32),
    )(meta, xs, w1, w3, w2)


def _make_combine(T, H, nslot):
    tpw = T // NW
    tch = 32
    mesh = plsc.VectorSubcoreMesh(core_axis_name="c", subcore_axis_name="s")

    @functools.partial(
        pl.kernel,
        out_type=jax.ShapeDtypeStruct((T, H), jnp.float32),
        mesh=mesh,
        scratch_types=[
            pltpu.VMEM((tch,), jnp.int32),
            pltpu.VMEM((tch,), jnp.int32),
            pltpu.VMEM((tch,), jnp.float32),
            pltpu.VMEM((tch,), jnp.float32),
            pltpu.VMEM((tch, H), jnp.float32),
            pltpu.VMEM((tch, H), jnp.float32),
            pltpu.VMEM((tch, H), jnp.float32),
            pltpu.SemaphoreType.DMA,
            pltpu.SemaphoreType.DMA,
        ],
    )
    def combine(ys_hbm, slots_hbm, w_hbm, out_hbm,
                idx0, idx1, w0v, w1v, g0, g1, outv, sem0, sem1):
        wid = lax.axis_index("s") * 2 + lax.axis_index("c")
        base = wid * tpw

        def bcast16(vec, lane):
            dn = lax.GatherDimensionNumbers(
                offset_dims=(), collapsed_slice_dims=(0,), start_index_map=(0,))
            idx = jnp.full((16, 1), lane, jnp.int32)
            return lax.gather(vec, idx, dn, (1,),
                              mode=lax.GatherScatterMode.PROMISE_IN_BOUNDS)

        for i in range(tpw // tch):
            tb = base + i * tch
            pltpu.sync_copy(slots_hbm.at[pl.ds(tb, tch)], idx0)
            pltpu.sync_copy(slots_hbm.at[pl.ds(T + tb, tch)], idx1)
            c0 = pltpu.async_copy(ys_hbm.at[idx0], g0, sem0)
            c1 = pltpu.async_copy(ys_hbm.at[idx1], g1, sem1)
            pltpu.sync_copy(w_hbm.at[pl.ds(tb, tch)], w0v)
            pltpu.sync_copy(w_hbm.at[pl.ds(T + tb, tch)], w1v)
            c0.wait()
            c1.wait()
            for t in range(tch):
                grp = (t // 16) * 16
                w0s = bcast16(w0v[pl.ds(grp, 16)], t % 16)
                w1s = bcast16(w1v[pl.ds(grp, 16)], t % 16)

                @plsc.parallel_loop(0, H // 16, unroll=8)
                def col(cc, t=t, w0s=w0s, w1s=w1s):
                    sl = pl.ds(cc * 16, 16)
                    outv[t, sl] = w0s * g0[t, sl] + w1s * g1[t, sl]

            pltpu.sync_copy(outv, out_hbm.at[pl.ds(tb, tch)])

    return combine


def kernel(hidden_states, gate_w, w1, w2, w3):
    B, S, H = hidden_states.shape
    E = gate_w.shape[0]
    T = B * S
    A = 2 * T
    nslot = ((A + (E - 1) * (TILE - 1)) + TILE - 1) // TILE * TILE
    ntiles = nslot // TILE
    meta_len = ((ntiles + 1) + 63) // 64 * 64

    x = hidden_states.reshape(T, H)
    topw, slots, meta, aux, xb = _router(x, gate_w.T, nslot, ntiles, meta_len)
    slots_flat = slots.reshape(A)
    xb32 = lax.bitcast_convert_type(xb.reshape(T, H // 2, 2), jnp.int32)
    xs32 = _make_dispatch(T, H // 2, A, nslot)(xb32, slots_flat)
    xs = lax.bitcast_convert_type(xs32, jnp.bfloat16).reshape(nslot, H)
    ys = _gemm(meta, xs, w1, w3, w2, nslot, ntiles)
    out = _make_combine(T, H, nslot)(ys, slots_flat, topw.reshape(A))
    return out.reshape(B, S, H), aux[0, 0]


# speedup vs baseline: 2.2150x; 2.2150x over previous
import functools

import jax
import jax.numpy as jnp
from jax import lax
from jax.experimental import pallas as pl
from jax.experimental.pallas import tpu as pltpu
from jax.experimental.pallas import tpu_sc as plsc

TILE = 128
NW = 32


def _router_body(nslot, ntiles, x_ref, gwt_ref, topw_ref, slots_ref, meta_ref,
                 aux_ref):
    T, H = x_ref.shape
    E = gwt_ref.shape[1]
    x = x_ref[...]
    logits = jnp.dot(x, gwt_ref[...], preferred_element_type=jnp.float32)
    RB = 128
    m = jnp.max(logits, axis=1, keepdims=True)
    ex = jnp.exp(logits - m)
    probs = ex / jnp.sum(ex, axis=1, keepdims=True)

    e_iota = lax.broadcasted_iota(jnp.int32, (T, E), 1)
    p1 = jnp.max(probs, axis=1, keepdims=True)
    first1 = jnp.min(jnp.where(probs == p1, e_iota, E), axis=1, keepdims=True)
    masked = jnp.where(e_iota == first1, -jnp.inf, probs)
    p2 = jnp.max(masked, axis=1, keepdims=True)
    first2 = jnp.min(jnp.where(masked == p2, e_iota, E), axis=1, keepdims=True)
    s = p1 + p2
    topw_ref[0, :] = (p1 / s).reshape(T)
    topw_ref[1, :] = (p2 / s).reshape(T)

    oh1 = (e_iota == first1).astype(jnp.float32)
    oh2 = (e_iota == first2).astype(jnp.float32)
    ohs = jnp.concatenate([oh1, oh2], 0)
    ri = lax.broadcasted_iota(jnp.int32, (RB, RB), 0)
    ci = lax.broadcasted_iota(jnp.int32, (RB, RB), 1)
    ls = (ri > ci).astype(jnp.float32)
    off = jnp.zeros((1, E), jnp.float32)
    rank_parts = []
    for b in range(2 * T // RB):
        blk = lax.slice(ohs, (b * RB, 0), ((b + 1) * RB, E))
        rblk = jnp.dot(ls, blk, preferred_element_type=jnp.float32) + off
        rank_parts.append(jnp.sum(rblk * blk, axis=1))
        off = off + jnp.sum(blk, axis=0, keepdims=True)
    ranks = jnp.concatenate(rank_parts, 0)

    cnt = off[0]
    cnt_pad = jnp.ceil(cnt / TILE) * TILE
    ri16 = lax.broadcasted_iota(jnp.int32, (E, E), 0)
    ci16 = lax.broadcasted_iota(jnp.int32, (E, E), 1)
    incl = jnp.sum(jnp.where(ci16 <= ri16, cnt_pad[None, :], 0.0), axis=1)
    excl = incl - cnt_pad
    slot_f = ranks + jnp.sum(ohs * excl[None, :], axis=1)
    slots_ref[0, :] = lax.slice(slot_f, (0,), (T,)).astype(jnp.int32)
    slots_ref[1, :] = lax.slice(slot_f, (T,), (2 * T,)).astype(jnp.int32)

    total = jnp.sum(cnt_pad)
    used = total / TILE
    js = lax.broadcasted_iota(jnp.int32, (ntiles, 1), 0).astype(jnp.float32) * TILE
    slot_eff = jnp.minimum(js, total - TILE)
    te = jnp.sum((slot_eff >= incl[None, :]).astype(jnp.int32), axis=1)

    nzf = (cnt_pad > 0).astype(jnp.float32)
    ordinal_e = jnp.sum(jnp.where(ci16 <= ri16, nzf[None, :], 0.0), axis=1) - 1.0
    nord = jnp.sum(nzf)
    te_oh = te[:, None] == lax.broadcasted_iota(jnp.int32, (ntiles, E), 1)
    ordt = jnp.sum(jnp.where(te_oh, ordinal_e[None, :], 0.0), axis=1)
    oo = lax.broadcasted_iota(jnp.int32, (E, E), 0).astype(jnp.float32)
    sel_oe = jnp.logical_and(ordinal_e[None, :] == oo, nzf[None, :] > 0.0)
    e_colf = lax.broadcasted_iota(jnp.int32, (E, E), 1).astype(jnp.float32)
    eord = jnp.sum(jnp.where(sel_oe, e_colf, 0.0), axis=1)

    pad = meta_ref.shape[1] - (2 * ntiles + 2 + E)
    meta_ref[0, :] = jnp.concatenate(
        [te, used.astype(jnp.int32).reshape(1),
         nord.astype(jnp.int32).reshape(1),
         ordt.astype(jnp.int32), eord.astype(jnp.int32),
         jnp.zeros((pad,), jnp.int32)], 0)

    meanprob = jnp.mean(probs, axis=0)
    aux_ref[...] = jnp.reshape(jnp.sum(cnt * meanprob) * E / T, (1, 1))


def _router(x, gwt, nslot, ntiles, meta_len):
    T, _ = x.shape
    return pl.pallas_call(
        functools.partial(_router_body, nslot, ntiles),
        out_shape=(
            jax.ShapeDtypeStruct((2, T), jnp.float32),
            jax.ShapeDtypeStruct((2, T), jnp.int32),
            jax.ShapeDtypeStruct((1, meta_len), jnp.int32),
            jax.ShapeDtypeStruct((1, 1), jnp.float32),
        ),
    )(x, gwt)


def _make_dispatch(T, H, A, nslot):
    tpw = T // NW
    mesh = plsc.VectorSubcoreMesh(core_axis_name="c", subcore_axis_name="s")

    @functools.partial(
        pl.kernel,
        out_type=jax.ShapeDtypeStruct((nslot, H), jnp.float32),
        mesh=mesh,
        scratch_types=[
            pltpu.VMEM((tpw,), jnp.int32),
            pltpu.VMEM((tpw,), jnp.int32),
            pltpu.VMEM((tpw, H), jnp.float32),
            pltpu.SemaphoreType.DMA,
            pltpu.SemaphoreType.DMA,
        ],
    )
    def dispatch(x_hbm, slots_hbm, xs_hbm, idx0, idx1, rows_v, sem0, sem1):
        wid = lax.axis_index("s") * 2 + lax.axis_index("c")
        tb = wid * tpw
        pltpu.sync_copy(slots_hbm.at[pl.ds(tb, tpw)], idx0)
        pltpu.sync_copy(slots_hbm.at[pl.ds(T + tb, tpw)], idx1)
        pltpu.sync_copy(x_hbm.at[pl.ds(tb, tpw)], rows_v)
        c0 = pltpu.async_copy(rows_v, xs_hbm.at[idx0], sem0)
        c1 = pltpu.async_copy(rows_v, xs_hbm.at[idx1], sem1)
        c0.wait()
        c1.wait()

    return dispatch


def _gemm_body(ntiles, meta_ref, xs_ref, w1_hbm, w3_hbm, w2_hbm, ys_ref,
               w1r, w3r, w2r, sem):
    j = pl.program_id(0)
    ORD = ntiles + 2
    EORD = 2 * ntiles + 2
    used = meta_ref[0, ntiles]
    nord = meta_ref[0, ntiles + 1]
    o = meta_ref[0, ORD + j]
    o_prev = meta_ref[0, ORD + jnp.maximum(j - 1, 0)]
    slot = lax.rem(o, 2)

    def fetch(oo, sl):
        e = meta_ref[0, EORD + oo]
        pltpu.make_async_copy(w1_hbm.at[e], w1r.at[sl], sem.at[sl]).start()
        pltpu.make_async_copy(w3_hbm.at[e], w3r.at[sl], sem.at[sl]).start()
        pltpu.make_async_copy(w2_hbm.at[e], w2r.at[sl], sem.at[sl]).start()

    def drain(sl):
        pltpu.make_async_copy(w1_hbm.at[0], w1r.at[sl], sem.at[sl]).wait()
        pltpu.make_async_copy(w3_hbm.at[0], w3r.at[sl], sem.at[sl]).wait()
        pltpu.make_async_copy(w2_hbm.at[0], w2r.at[sl], sem.at[sl]).wait()

    @pl.when(j == 0)
    def _():
        fetch(0, 0)

    first = jnp.logical_or(j == 0, o != o_prev)

    @pl.when(jnp.logical_and(first, j < used))
    def _():
        drain(slot)

        @pl.when(o + 1 < nord)
        def _():
            fetch(o + 1, lax.rem(o + 1, 2))

    @pl.when(j < used)
    def _():
        xt = xs_ref[...]
        h1 = jnp.dot(xt, w1r[slot], preferred_element_type=jnp.float32)
        h3 = jnp.dot(xt, w3r[slot], preferred_element_type=jnp.float32)
        act = h1 * lax.logistic(h1) * h3
        ys_ref[...] = jnp.dot(act, w2r[slot], preferred_element_type=jnp.float32)


def _gemm(meta, xs, w1, w3, w2, nslot, ntiles):
    E, H, F = w1.shape
    grid_spec = pltpu.PrefetchScalarGridSpec(
        num_scalar_prefetch=1,
        grid=(ntiles,),
        in_specs=[
            pl.BlockSpec((TILE, H),
                         lambda j, m: (jnp.minimum(j, m[0, ntiles] - 1), 0)),
            pl.BlockSpec(memory_space=pl.ANY),
            pl.BlockSpec(memory_space=pl.ANY),
            pl.BlockSpec(memory_space=pl.ANY),
        ],
        out_specs=pl.BlockSpec(
            (TILE, H), lambda j, m: (jnp.minimum(j, m[0, ntiles] - 1), 0)),
        scratch_shapes=[
            pltpu.VMEM((2, H, F), jnp.float32),
            pltpu.VMEM((2, H, F), jnp.float32),
            pltpu.VMEM((2, F, H), jnp.float32),
            pltpu.SemaphoreType.DMA((2,)),
        ],
    )
    return pl.pallas_call(
        functools.partial(_gemm_body, ntiles),
        grid_spec=grid_spec,
        out_shape=jax.ShapeDtypeStruct((nslot, H), jnp.float32),
    )(meta, xs, w1, w3, w2)


def _make_combine(T, H, nslot):
    tpw = T // NW
    tch = 32
    mesh = plsc.VectorSubcoreMesh(core_axis_name="c", subcore_axis_name="s")

    @functools.partial(
        pl.kernel,
        out_type=jax.ShapeDtypeStruct((T, H), jnp.float32),
        mesh=mesh,
        scratch_types=[
            pltpu.VMEM((tch,), jnp.int32),
            pltpu.VMEM((tch,), jnp.int32),
            pltpu.VMEM((tch,), jnp.float32),
            pltpu.VMEM((tch,), jnp.float32),
            pltpu.VMEM((tch, H), jnp.float32),
            pltpu.VMEM((tch, H), jnp.float32),
            pltpu.VMEM((tch, H), jnp.float32),
            pltpu.SemaphoreType.DMA,
            pltpu.SemaphoreType.DMA,
        ],
    )
    def combine(ys_hbm, slots_hbm, w_hbm, out_hbm,
                idx0, idx1, w0v, w1v, g0, g1, outv, sem0, sem1):
        wid = lax.axis_index("s") * 2 + lax.axis_index("c")
        base = wid * tpw

        def bcast16(vec, lane):
            dn = lax.GatherDimensionNumbers(
                offset_dims=(), collapsed_slice_dims=(0,), start_index_map=(0,))
            idx = jnp.full((16, 1), lane, jnp.int32)
            return lax.gather(vec, idx, dn, (1,),
                              mode=lax.GatherScatterMode.PROMISE_IN_BOUNDS)

        for i in range(tpw // tch):
            tb = base + i * tch
            pltpu.sync_copy(slots_hbm.at[pl.ds(tb, tch)], idx0)
            pltpu.sync_copy(slots_hbm.at[pl.ds(T + tb, tch)], idx1)
            c0 = pltpu.async_copy(ys_hbm.at[idx0], g0, sem0)
            c1 = pltpu.async_copy(ys_hbm.at[idx1], g1, sem1)
            pltpu.sync_copy(w_hbm.at[pl.ds(tb, tch)], w0v)
            pltpu.sync_copy(w_hbm.at[pl.ds(T + tb, tch)], w1v)
            c0.wait()
            c1.wait()
            for t in range(tch):
                grp = (t // 16) * 16
                w0s = bcast16(w0v[pl.ds(grp, 16)], t % 16)
                w1s = bcast16(w1v[pl.ds(grp, 16)], t % 16)

                @plsc.parallel_loop(0, H // 16, unroll=8)
                def col(cc, t=t, w0s=w0s, w1s=w1s):
                    sl = pl.ds(cc * 16, 16)
                    outv[t, sl] = w0s * g0[t, sl] + w1s * g1[t, sl]

            pltpu.sync_copy(outv, out_hbm.at[pl.ds(tb, tch)])

    return combine


def kernel(hidden_states, gate_w, w1, w2, w3):
    B, S, H = hidden_states.shape
    E = gate_w.shape[0]
    T = B * S
    A = 2 * T
    nslot = ((A + (E - 1) * (TILE - 1)) + TILE - 1) // TILE * TILE
    ntiles = nslot // TILE
    meta_len = ((2 * ntiles + 2 + E) + 63) // 64 * 64

    x = hidden_states.reshape(T, H)
    topw, slots, meta, aux = _router(x, gate_w.T, nslot, ntiles, meta_len)
    slots_flat = slots.reshape(A)
    xs = _make_dispatch(T, H, A, nslot)(x, slots_flat)
    ys = _gemm(meta, xs, w1, w3, w2, nslot, ntiles)
    out = _make_combine(T, H, nslot)(ys, slots_flat, topw.reshape(A))
    return out.reshape(B, S, H), aux[0, 0]
